# baseline (speedup 1.0000x reference)
import functools

import jax
import jax.numpy as jnp
from jax import lax
from jax.experimental import pallas as pl
from jax.experimental.pallas import tpu as pltpu
from jax.experimental.pallas import tpu_sc as plsc

_NC = 2
_NS = 16
_L = 16
_DIM = 64
_NCARD_IDS = 52
_NRANK = 13
_NSUIT = 4
_K = 5


@functools.lru_cache(maxsize=None)
def _make_sc_kernel(n_rows):
    nw = _NC * _NS
    rows_per_w = n_rows // nw
    groups = rows_per_w // _L
    assert rows_per_w * nw == n_rows and groups * _L == rows_per_w
    assert groups % 2 == 0

    mesh = plsc.VectorSubcoreMesh(core_axis_name="c", subcore_axis_name="s")

    @functools.partial(
        pl.kernel,
        mesh=mesh,
        out_type=(
            jax.ShapeDtypeStruct((n_rows, _DIM), jnp.float32),
            jax.ShapeDtypeStruct((_NC * 56, 128), jnp.float32),
        ),
        compiler_params=pltpu.CompilerParams(needs_layout_passes=False),
        scratch_types=[
            pltpu.VMEM((56, 128), jnp.float32),
            pltpu.VMEM((_NCARD_IDS, _DIM), jnp.float32),
            pltpu.VMEM((_NRANK, _DIM), jnp.float32),
            pltpu.VMEM((_NSUIT, _DIM), jnp.float32),
        ] + [
            pltpu.VMEM((rows_per_w,), jnp.int32)
            for _ in range(_K)
        ] + [
            pltpu.VMEM((_L, 128), jnp.float32)
            for _ in range(2 * _K)
        ] + [
            pltpu.VMEM((_L, _DIM), jnp.float32),
            pltpu.VMEM((_L, _DIM), jnp.float32),
            pltpu.SemaphoreType.DMA,
            pltpu.SemaphoreType.DMA,
            pltpu.SemaphoreType.DMA,
        ],
    )
    def sc_kernel(idx_hbm, card_hbm, rank_hbm, suit_hbm, out_hbm, t_hbm,
                  t_v, card_v, rank_v, suit_v, *rest):
        idx_vs = rest[:_K]
        rows_v = [rest[_K + p * _K:_K + (p + 1) * _K] for p in range(2)]
        out_v = rest[3 * _K:3 * _K + 2]
        gsem = rest[3 * _K + 2:3 * _K + 4]
        osem = rest[3 * _K + 4]

        cid = lax.axis_index("c")
        sid = lax.axis_index("s")
        wid = sid * _NC + cid
        row0 = wid * rows_per_w

        @pl.when(sid == 0)
        def _build_table():
            pltpu.sync_copy(card_hbm, card_v)
            pltpu.sync_copy(rank_hbm, rank_v)
            pltpu.sync_copy(suit_hbm, suit_v)
            for i in range(_NCARD_IDS):
                r, s = i // 4, i % 4
                for c in range(0, _DIM, _L):
                    t_v[i, pl.ds(c, _L)] = (card_v[i, pl.ds(c, _L)]
                                            + rank_v[r, pl.ds(c, _L)]
                                            + suit_v[s, pl.ds(c, _L)])
            pltpu.sync_copy(t_v, t_hbm.at[pl.ds(cid * 56, 56)])

        for j in range(_K):
            pltpu.sync_copy(idx_hbm.at[pl.ds(j * n_rows + row0, rows_per_w)],
                            idx_vs[j])
        bias = cid * 56
        for j in range(_K):
            for b in range(0, rows_per_w, _L):
                idx_vs[j][pl.ds(b, _L)] = idx_vs[j][pl.ds(b, _L)] + bias

        plsc.subcore_barrier()

        def start_gathers(p, g):
            for j in range(_K):
                pltpu.async_copy(
                    t_hbm.at[idx_vs[j].at[pl.ds(g * _L, _L)]],
                    rows_v[p][j], gsem[p])

        def wait_gathers(p):
            for j in range(_K):
                pltpu.make_async_copy(
                    t_hbm.at[pl.ds(0, _L)], rows_v[p][j], gsem[p]).wait()

        def out_slice(g):
            return out_hbm.at[pl.ds(row0 + g * _L, _L)]

        def phase(p, g, i):
            @pl.when(i > 0)
            def _reclaim():
                pltpu.make_async_copy(out_v[p], out_slice(g - 2), osem).wait()
            wait_gathers(p)
            for r in range(_L):
                for c in range(0, _DIM, _L):
                    acc = rows_v[p][0][r, pl.ds(c, _L)]
                    for j in range(1, _K):
                        acc = acc + rows_v[p][j][r, pl.ds(c, _L)]
                    out_v[p][r, pl.ds(c, _L)] = acc
            pltpu.async_copy(out_v[p], out_slice(g), osem)
            @pl.when(g + 2 < groups)
            def _prefetch():
                start_gathers(p, g + 2)

        start_gathers(0, 0)
        start_gathers(1, 1)

        def loop_body(i, carry):
            phase(0, 2 * i, i)
            phase(1, 2 * i + 1, i)
            return carry

        lax.fori_loop(0, groups // 2, loop_body, 0)

        pltpu.make_async_copy(out_v[0], out_slice(groups - 2), osem).wait()
        pltpu.make_async_copy(out_v[1], out_slice(groups - 1), osem).wait()

    return sc_kernel


def kernel(input, card, rank, suit):
    n, _ = input.shape
    idx = input.astype(jnp.int32).T.reshape(-1)
    out, _unused_t = _make_sc_kernel(n)(idx, card, rank, suit)
    return out


# speedup vs baseline: 2.1439x; 2.1439x over previous
import functools

import jax
import jax.numpy as jnp
from jax import lax
from jax.experimental import pallas as pl
from jax.experimental.pallas import tpu as pltpu
from jax.experimental.pallas import tpu_sc as plsc

_NC = 2
_NS = 16
_L = 16
_DIM = 64
_NCARD_IDS = 52
_NRANK = 13
_NSUIT = 4
_K = 5
_DBLK = 16


@functools.lru_cache(maxsize=None)
def _make_sc_kernel(n_rows):
    nw = _NC * _NS
    rows_per_w = n_rows // nw
    groups = rows_per_w // _L
    assert rows_per_w * nw == n_rows and groups * _L == rows_per_w

    mesh = plsc.VectorSubcoreMesh(core_axis_name="c", subcore_axis_name="s")

    @functools.partial(
        pl.kernel,
        mesh=mesh,
        out_type=jax.ShapeDtypeStruct((n_rows, _DIM), jnp.float32),
        compiler_params=pltpu.CompilerParams(needs_layout_passes=False),
        scratch_types=[
            pltpu.VMEM((_NCARD_IDS * _DIM,), jnp.float32),
            pltpu.VMEM((_NCARD_IDS, _DIM), jnp.float32),
            pltpu.VMEM((_NRANK, _DIM), jnp.float32),
            pltpu.VMEM((_NSUIT, _DIM), jnp.float32),
        ] + [
            pltpu.VMEM((rows_per_w,), jnp.int32)
            for _ in range(_K)
        ] + [
            pltpu.VMEM((rows_per_w, _DIM), jnp.float32),
        ],
    )
    def sc_kernel(idx_hbm, card_hbm, rank_hbm, suit_hbm, out_hbm,
                  t_v, card_v, rank_v, suit_v, *idx_and_out):
        idx_vs = idx_and_out[:_K]
        out_v = idx_and_out[_K]
        wid = lax.axis_index("s") * _NC + lax.axis_index("c")
        row0 = wid * rows_per_w

        pltpu.sync_copy(card_hbm, card_v)
        pltpu.sync_copy(rank_hbm, rank_v)
        pltpu.sync_copy(suit_hbm, suit_v)
        for j in range(_K):
            pltpu.sync_copy(idx_hbm.at[pl.ds(j * n_rows + row0, rows_per_w)],
                            idx_vs[j])

        for i in range(_NCARD_IDS):
            r, s = i // 4, i % 4
            for c in range(0, _DIM, _L):
                t_v[pl.ds(i * _DIM + c, _L)] = (card_v[i, pl.ds(c, _L)]
                                                + rank_v[r, pl.ds(c, _L)]
                                                + suit_v[s, pl.ds(c, _L)])

        lanes = lax.iota(jnp.int32, _L)

        @plsc.parallel_loop(0, groups, unroll=2)
        def group_body(g):
            n0 = g * _L
            rowvec = n0 + lanes
            idb = [idx_vs[j][pl.ds(n0, _L)] * _DIM for j in range(_K)]
            for d0 in range(0, _DIM, _DBLK):
                accs = []
                for d in range(d0, d0 + _DBLK):
                    cvec = (lanes + d) & (_DIM - 1)
                    acc = plsc.load_gather(t_v, [idb[0] + cvec])
                    for j in range(1, _K):
                        acc = acc + plsc.load_gather(t_v, [idb[j] + cvec])
                    accs.append((cvec, acc))
                for cvec, acc in accs:
                    plsc.store_scatter(out_v, [rowvec, cvec], acc)

        pltpu.sync_copy(out_v, out_hbm.at[pl.ds(row0, rows_per_w)])

    return sc_kernel


def kernel(input, card, rank, suit):
    n, _ = input.shape
    idx = input.astype(jnp.int32).T.reshape(-1)
    return _make_sc_kernel(n)(idx, card, rank, suit)


# speedup vs baseline: 2.2249x; 1.0378x over previous
import functools

import jax
import jax.numpy as jnp
from jax import lax
from jax.experimental import pallas as pl
from jax.experimental.pallas import tpu as pltpu
from jax.experimental.pallas import tpu_sc as plsc

_NC = 2
_NS = 16
_L = 16
_DIM = 64
_NCARD_IDS = 52
_NRANK = 13
_NSUIT = 4
_K = 5
_DBLK = 16


@functools.lru_cache(maxsize=None)
def _make_sc_kernel(n_rows):
    nw = _NC * _NS
    rows_per_w = n_rows // nw
    groups = rows_per_w // _L
    assert rows_per_w * nw == n_rows and groups * _L == rows_per_w

    mesh = plsc.VectorSubcoreMesh(core_axis_name="c", subcore_axis_name="s")

    @functools.partial(
        pl.kernel,
        mesh=mesh,
        out_type=jax.ShapeDtypeStruct((n_rows, _DIM), jnp.float32),
        compiler_params=pltpu.CompilerParams(needs_layout_passes=False),
        scratch_types=[
            pltpu.VMEM((_NCARD_IDS * _DIM,), jnp.float32),
            pltpu.VMEM((_NCARD_IDS, _DIM), jnp.float32),
            pltpu.VMEM((_NRANK, _DIM), jnp.float32),
            pltpu.VMEM((_NSUIT, _DIM), jnp.float32),
        ] + [
            pltpu.VMEM((rows_per_w,), jnp.int32)
            for _ in range(_K)
        ] + [
            pltpu.VMEM((rows_per_w, _DIM), jnp.float32),
        ],
    )
    def sc_kernel(idx_hbm, card_hbm, rank_hbm, suit_hbm, out_hbm,
                  t_v, card_v, rank_v, suit_v, *idx_and_out):
        idx_vs = idx_and_out[:_K]
        out_v = idx_and_out[_K]
        wid = lax.axis_index("s") * _NC + lax.axis_index("c")
        row0 = wid * rows_per_w

        pltpu.sync_copy(card_hbm, card_v)
        pltpu.sync_copy(rank_hbm, rank_v)
        pltpu.sync_copy(suit_hbm, suit_v)
        for j in range(_K):
            pltpu.sync_copy(idx_hbm.at[pl.ds(j * n_rows + row0, rows_per_w)],
                            idx_vs[j])

        for i in range(_NCARD_IDS):
            r, s = i // 4, i % 4
            for c in range(0, _DIM, _L):
                t_v[pl.ds(i * _DIM + c, _L)] = (card_v[i, pl.ds(c, _L)]
                                                + rank_v[r, pl.ds(c, _L)]
                                                + suit_v[s, pl.ds(c, _L)])

        lanes = lax.iota(jnp.int32, _L)

        @plsc.parallel_loop(0, groups)
        def group_body(g):
            n0 = g * _L
            rowvec = n0 + lanes
            idb = [idx_vs[j][pl.ds(n0, _L)] * _DIM for j in range(_K)]
            for d0 in range(0, _DIM, _DBLK):
                accs = []
                for d in range(d0, d0 + _DBLK):
                    cvec = (lanes + d) & (_DIM - 1)
                    g0, g1, g2, g3, g4 = [
                        plsc.load_gather(t_v, [idb[j] + cvec])
                        for j in range(_K)]
                    acc = (g0 + g1) + (g2 + g3) + g4
                    accs.append((cvec, acc))
                for cvec, acc in accs:
                    plsc.store_scatter(out_v, [rowvec, cvec], acc)

        pltpu.sync_copy(out_v, out_hbm.at[pl.ds(row0, rows_per_w)])

    return sc_kernel


def kernel(input, card, rank, suit):
    n, _ = input.shape
    idx = input.astype(jnp.int32).T.reshape(-1)
    return _make_sc_kernel(n)(idx, card, rank, suit)


# speedup vs baseline: 2.3370x; 1.0504x over previous
import functools

import jax
import jax.numpy as jnp
from jax import lax
from jax.experimental import pallas as pl
from jax.experimental.pallas import tpu as pltpu
from jax.experimental.pallas import tpu_sc as plsc

_NC = 2
_NS = 16
_L = 16
_DIM = 64
_NCARD_IDS = 52
_NRANK = 13
_NSUIT = 4
_K = 5
_DBLK = 8


@functools.lru_cache(maxsize=None)
def _make_sc_kernel(n_rows):
    nw = _NC * _NS
    rows_per_w = n_rows // nw
    groups = rows_per_w // _L
    assert rows_per_w * nw == n_rows and groups * _L == rows_per_w

    mesh = plsc.VectorSubcoreMesh(core_axis_name="c", subcore_axis_name="s")

    @functools.partial(
        pl.kernel,
        mesh=mesh,
        out_type=jax.ShapeDtypeStruct((n_rows, _DIM), jnp.float32),
        compiler_params=pltpu.CompilerParams(needs_layout_passes=False),
        scratch_types=[
            pltpu.VMEM((_NCARD_IDS * _DIM,), jnp.float32),
            pltpu.VMEM((_NCARD_IDS, _DIM), jnp.float32),
            pltpu.VMEM((_NRANK, _DIM), jnp.float32),
            pltpu.VMEM((_NSUIT, _DIM), jnp.float32),
        ] + [
            pltpu.VMEM((rows_per_w,), jnp.int32)
            for _ in range(_K)
        ] + [
            pltpu.VMEM((rows_per_w, _DIM), jnp.float32),
        ],
    )
    def sc_kernel(idx_hbm, card_hbm, rank_hbm, suit_hbm, out_hbm,
                  t_v, card_v, rank_v, suit_v, *idx_and_out):
        idx_vs = idx_and_out[:_K]
        out_v = idx_and_out[_K]
        wid = lax.axis_index("s") * _NC + lax.axis_index("c")
        row0 = wid * rows_per_w

        pltpu.sync_copy(card_hbm, card_v)
        pltpu.sync_copy(rank_hbm, rank_v)
        pltpu.sync_copy(suit_hbm, suit_v)
        for j in range(_K):
            pltpu.sync_copy(idx_hbm.at[pl.ds(j * n_rows + row0, rows_per_w)],
                            idx_vs[j])

        for i in range(_NCARD_IDS):
            r, s = i // 4, i % 4
            for c in range(0, _DIM, _L):
                t_v[pl.ds(i * _DIM + c, _L)] = (card_v[i, pl.ds(c, _L)]
                                                + rank_v[r, pl.ds(c, _L)]
                                                + suit_v[s, pl.ds(c, _L)])

        lanes = lax.iota(jnp.int32, _L)

        @plsc.parallel_loop(0, groups)
        def group_body(g):
            n0 = g * _L
            rowvec = n0 + lanes
            idb = [idx_vs[j][pl.ds(n0, _L)] * _DIM for j in range(_K)]
            for d0 in range(0, _DIM, _DBLK):
                accs = []
                for d in range(d0, d0 + _DBLK):
                    cvec = (lanes + d) & (_DIM - 1)
                    g0, g1, g2, g3, g4 = [
                        plsc.load_gather(t_v, [idb[j] + cvec])
                        for j in range(_K)]
                    acc = (g0 + g1) + (g2 + g3) + g4
                    accs.append((cvec, acc))
                for cvec, acc in accs:
                    plsc.store_scatter(out_v, [rowvec, cvec], acc)

        pltpu.sync_copy(out_v, out_hbm.at[pl.ds(row0, rows_per_w)])

    return sc_kernel


def kernel(input, card, rank, suit):
    n, _ = input.shape
    idx = input.astype(jnp.int32).T.reshape(-1)
    return _make_sc_kernel(n)(idx, card, rank, suit)
